# Initial kernel scaffold; baseline (speedup 1.0000x reference)
import jax
import jax.numpy as jnp
from jax.experimental import pallas as pl


def kernel(query_xyz, support_xyz, query_mask, support_mask, support_features, conv_w, bn_gamma, bn_beta):
    raise NotImplementedError("write your pallas kernel here")


# speedup vs baseline: 38.5990x; 38.5990x over previous
import math

import jax
import jax.numpy as jnp
from jax import lax
from jax.experimental import pallas as pl

B, N1, N2 = 4, 512, 2048
IN_C, OUT_C = 384, 512
RADIUS, NSAMPLE = 0.3, 32
FEAT_DIM = IN_C // 6
EPS = 1e-5
ALPHA0 = 100.0 / RADIUS
NEG_LOG1000_OVER_FD = -math.log(1000.0) / FEAT_DIM


def _pospool_body(qx_ref, sxT_ref, feat_ref, wT_ref, smask_ref, y_ref):
    f32 = jnp.float32
    qx = qx_ref[...]
    sxT = sxT_ref[...]

    d2 = jnp.zeros((N1, N2), f32)
    for d in range(3):
        diff = qx[:, d:d + 1] - sxT[d:d + 1, :]
        d2 = d2 + diff * diff
    valid = (d2 < RADIUS * RADIUS) & (smask_ref[...] > 0.0)
    v = jnp.where(valid, 1.0, 0.0).astype(f32)

    inc = v
    k = 1
    while k < N2:
        inc = inc + jnp.concatenate(
            [jnp.zeros((N1, k), f32), inc[:, :N2 - k]], axis=1)
        k *= 2
    m = v * jnp.where(inc <= float(NSAMPLE), 1.0, 0.0)
    cnt = jnp.sum(m, axis=1, keepdims=True)

    rowi = lax.broadcasted_iota(jnp.int32, (3 * FEAT_DIM, N2), 0)
    jrow = lax.rem(rowi, FEAT_DIM).astype(f32)
    alpha = ALPHA0 * jnp.exp(jrow * NEG_LOG1000_OVER_FD)
    srep = jnp.concatenate(
        [jnp.broadcast_to(sxT[d:d + 1, :], (FEAT_DIM, N2)) for d in range(3)],
        axis=0)
    phs = alpha * srep
    ssin, scos = jnp.sin(phs), jnp.cos(phs)

    def dup_rows(t):
        return jnp.concatenate(
            [t[0:FEAT_DIM], t[0:FEAT_DIM],
             t[FEAT_DIM:2 * FEAT_DIM], t[FEAT_DIM:2 * FEAT_DIM],
             t[2 * FEAT_DIM:], t[2 * FEAT_DIM:]], axis=0)

    feat = feat_ref[...]
    ta = feat * dup_rows(ssin)
    tb = feat * dup_rows(scos)

    dn = (((1,), (1,)), ((), ()))
    pa = lax.dot_general(m, ta, dn, preferred_element_type=f32)
    pb = lax.dot_general(m, tb, dn, preferred_element_type=f32)

    lanei = lax.broadcasted_iota(jnp.int32, (N1, 3 * FEAT_DIM), 1)
    jlane = lax.rem(lanei, FEAT_DIM).astype(f32)
    alphaq = ALPHA0 * jnp.exp(jlane * NEG_LOG1000_OVER_FD)
    qrep = jnp.concatenate(
        [jnp.broadcast_to(qx[:, d:d + 1], (N1, FEAT_DIM)) for d in range(3)],
        axis=1)
    phq = alphaq * qrep
    qs, qc = jnp.sin(phq), jnp.cos(phq)

    def dup_lanes(t):
        return jnp.concatenate(
            [t[:, 0:FEAT_DIM], t[:, 0:FEAT_DIM],
             t[:, FEAT_DIM:2 * FEAT_DIM], t[:, FEAT_DIM:2 * FEAT_DIM],
             t[:, 2 * FEAT_DIM:], t[:, 2 * FEAT_DIM:]], axis=1)

    qs4, qc4 = dup_lanes(qs), dup_lanes(qc)
    r = lax.rem(lax.broadcasted_iota(jnp.int32, (N1, IN_C), 1), 2 * FEAT_DIM)
    is_sin = r < FEAT_DIM
    x = jnp.where(is_sin, pa, pb)
    y = jnp.where(is_sin, -pb, pa)
    ofeat = (qc4 * x + qs4 * y) / cnt

    y_ref[...] = jnp.dot(ofeat, wT_ref[...], preferred_element_type=f32)


def _bn_body(y_ref, g_ref, b_ref, out_ref):
    y = y_ref[...]
    mean = jnp.mean(y, axis=0, keepdims=True)
    d = y - mean
    var = jnp.mean(d * d, axis=0, keepdims=True)
    o = d * lax.rsqrt(var + EPS) * g_ref[...] + b_ref[...]
    o = jnp.maximum(o, 0.0)
    for b in range(B):
        out_ref[b] = o[b * N1:(b + 1) * N1, :].T


def kernel(query_xyz, support_xyz, query_mask, support_mask,
           support_features, conv_w, bn_gamma, bn_beta):
    sxT = jnp.transpose(support_xyz, (0, 2, 1))
    wT = jnp.transpose(conv_w)
    y = pl.pallas_call(
        _pospool_body,
        grid=(B,),
        in_specs=[
            pl.BlockSpec((None, N1, 3), lambda b: (b, 0, 0)),
            pl.BlockSpec((None, 3, N2), lambda b: (b, 0, 0)),
            pl.BlockSpec((None, IN_C, N2), lambda b: (b, 0, 0)),
            pl.BlockSpec((IN_C, OUT_C), lambda b: (0, 0)),
            pl.BlockSpec((None, 1, N2), lambda b: (b, 0, 0)),
        ],
        out_specs=pl.BlockSpec((None, N1, OUT_C), lambda b: (b, 0, 0)),
        out_shape=jax.ShapeDtypeStruct((B, N1, OUT_C), jnp.float32),
    )(query_xyz, sxT, support_features, wT, support_mask[:, None, :])

    out = pl.pallas_call(
        _bn_body,
        out_shape=jax.ShapeDtypeStruct((B, OUT_C, N1), jnp.float32),
    )(y.reshape(B * N1, OUT_C), bn_gamma[None, :], bn_beta[None, :])
    return out
